# baseline (speedup 1.0000x reference)
import functools

import jax
import jax.numpy as jnp
from jax import lax
from jax.experimental import pallas as pl
from jax.experimental.pallas import tpu as pltpu
from jax.experimental.pallas import tpu_sc as plsc

L = 16


def _build_sc_gather(n_b: int, n_l: int, n_rows: int, d: int):
    info = plsc.get_sparse_core_info()
    nc, ns = info.num_cores, info.num_subcores
    nw = nc * ns

    bw = n_b // nw
    assert bw * nw == n_b
    lt_n = n_l // 8
    assert lt_n * 8 == n_l
    bc_w = 128
    s_n = bw // bc_w
    assert s_n * bc_w == bw
    groups = bc_w // L
    pairs = 8 * groups

    mesh = plsc.VectorSubcoreMesh(core_axis_name="c", subcore_axis_name="s")

    @functools.partial(
        pl.kernel,
        mesh=mesh,
        compiler_params=pltpu.CompilerParams(needs_layout_passes=False),
        out_type=jax.ShapeDtypeStruct((d, n_l, n_b), jnp.float32),
        scratch_types=[
            pltpu.VMEM((n_rows * d,), jnp.float32),
            pltpu.VMEM((bc_w, n_l), jnp.int32),
            pltpu.VMEM((d, 8, bc_w), jnp.float32),
            pltpu.VMEM((d, 8, bc_w), jnp.float32),
            pltpu.SemaphoreType.DMA,
            pltpu.SemaphoreType.DMA,
        ],
    )
    def sc_gather(idx_hbm, tbl_hbm, out_hbm, tbl_v, idx_blk,
                  out_v0, out_v1, sout0, sout1):
        sout = (sout0, sout1)
        out_v = (out_v0, out_v1)
        wid = lax.axis_index("s") * nc + lax.axis_index("c")
        b0w = wid * bw
        pltpu.sync_copy(tbl_hbm, tbl_v)
        lane = lax.iota(jnp.int32, L)

        def do_chunk(s, b0, lt, gi, p):
            @pl.when(gi >= 2)
            def _wait_out():
                pltpu.make_async_copy(
                    out_v[p],
                    out_hbm.at[:, pl.ds(lt * 8, 8), pl.ds(b0, bc_w)],
                    sout[p],
                ).wait()

            @plsc.parallel_loop(0, pairs, 1, unroll=2)
            def group_body(pr):
                lr = pr // groups
                g = pr % groups
                bvec = g * L + lane
                lvec = lane * 0 + (lt * 8 + lr)
                idx16 = plsc.load_gather(idx_blk, [bvec, lvec])
                addr = idx16 * d
                for f in range(d):
                    vals = plsc.load_gather(tbl_v, [addr + f])
                    out_v[p][f, lr, pl.ds(g * L, L)] = vals

            pltpu.async_copy(
                out_v[p],
                out_hbm.at[:, pl.ds(lt * 8, 8), pl.ds(b0, bc_w)],
                sout[p],
            )

        for s in range(s_n):
            b0 = b0w + s * bc_w
            pltpu.sync_copy(idx_hbm.at[pl.ds(b0, bc_w), :], idx_blk)

            def pair_body(cj, _, s=s, b0=b0):
                for b in range(2):
                    lt = cj * 2 + b
                    gi = s * lt_n + lt
                    do_chunk(s, b0, lt, gi, (s + b) % 2)
                return 0

            lax.fori_loop(0, lt_n // 2, pair_body, 0, unroll=False)
            do_chunk(s, b0, lt_n - 1, s * lt_n + lt_n - 1, s % 2)

        for p in range(2):
            pltpu.make_async_copy(
                out_v[p],
                out_hbm.at[:, pl.ds(0, 8), pl.ds(b0w, bc_w)],
                sout[p],
            ).wait()

    return sc_gather


def kernel(indices, features):
    b, hl = indices.shape
    v, d = features.shape
    tbl_flat = features.reshape(-1)
    fn = _build_sc_gather(b, hl, v, d)
    out_t = fn(indices.astype(jnp.int32), tbl_flat)
    return jnp.transpose(out_t, (2, 1, 0))


# speedup vs baseline: 1.7708x; 1.7708x over previous
import functools

import jax
import jax.numpy as jnp
from jax import lax
from jax.experimental import pallas as pl
from jax.experimental.pallas import tpu as pltpu
from jax.experimental.pallas import tpu_sc as plsc

L = 16


def _build_sc_gather(n_b: int, n_l: int, n_rows: int, d: int):
    info = plsc.get_sparse_core_info()
    nc, ns = info.num_cores, info.num_subcores
    nw = nc * ns

    bw = n_b // nw
    assert bw * nw == n_b
    lt_n = n_l // 8
    assert lt_n * 8 == n_l
    groups = bw // L
    pairs = 8 * groups

    mesh = plsc.VectorSubcoreMesh(core_axis_name="c", subcore_axis_name="s")

    @functools.partial(
        pl.kernel,
        mesh=mesh,
        compiler_params=pltpu.CompilerParams(needs_layout_passes=False),
        out_type=jax.ShapeDtypeStruct((d, n_l, n_b), jnp.float32),
        scratch_types=[
            pltpu.VMEM((n_rows * d,), jnp.float32),
            pltpu.VMEM((8, bw), jnp.int32),
            pltpu.VMEM((8, bw), jnp.int32),
            pltpu.VMEM((d, 8, bw), jnp.float32),
            pltpu.VMEM((d, 8, bw), jnp.float32),
            pltpu.SemaphoreType.DMA,
            pltpu.SemaphoreType.DMA,
            pltpu.SemaphoreType.DMA,
            pltpu.SemaphoreType.DMA,
        ],
    )
    def sc_gather(idx_hbm, tbl_hbm, out_hbm, tbl_v, idx_v0, idx_v1,
                  out_v0, out_v1, sin0, sin1, sout0, sout1):
        sin = (sin0, sin1)
        sout = (sout0, sout1)
        idx_v = (idx_v0, idx_v1)
        out_v = (out_v0, out_v1)
        wid = lax.axis_index("s") * nc + lax.axis_index("c")
        b0w = wid * bw
        pltpu.sync_copy(tbl_hbm, tbl_v)

        def in_slice(lt):
            return idx_hbm.at[pl.ds(lt * 8, 8), pl.ds(b0w, bw)]

        def out_slice(lt):
            return out_hbm.at[:, pl.ds(lt * 8, 8), pl.ds(b0w, bw)]

        def do_chunk(lt, p):
            pltpu.make_async_copy(in_slice(lt), idx_v[p], sin[p]).wait()

            @pl.when(lt >= 2)
            def _wait_out():
                pltpu.make_async_copy(out_v[p], out_slice(lt), sout[p]).wait()

            @plsc.parallel_loop(0, pairs, 1, unroll=4)
            def group_body(pr):
                lr = pr // groups
                g = pr % groups
                idx16 = idx_v[p][lr, pl.ds(g * L, L)]
                addr = idx16 * d
                for f in range(d):
                    vals = plsc.load_gather(tbl_v, [addr + f])
                    out_v[p][f, lr, pl.ds(g * L, L)] = vals

            pltpu.async_copy(out_v[p], out_slice(lt), sout[p])

            @pl.when(lt + 2 < lt_n)
            def _next_in():
                pltpu.async_copy(in_slice(lt + 2), idx_v[p], sin[p])

        for p in range(2):
            pltpu.async_copy(in_slice(p), idx_v[p], sin[p])

        def pair_body(cj, _):
            for b in range(2):
                do_chunk(cj * 2 + b, b)
            return 0

        lax.fori_loop(0, lt_n // 2, pair_body, 0, unroll=False)
        do_chunk(lt_n - 1, (lt_n - 1) % 2)

        for p in range(2):
            pltpu.make_async_copy(out_v[p], out_slice(p), sout[p]).wait()

    return sc_gather


def kernel(indices, features):
    b, hl = indices.shape
    v, d = features.shape
    idx_t = jnp.transpose(indices.astype(jnp.int32))
    tbl_flat = features.reshape(-1)
    fn = _build_sc_gather(b, hl, v, d)
    out_t = fn(idx_t, tbl_flat)
    return jnp.transpose(out_t, (2, 1, 0))


# speedup vs baseline: 1.8152x; 1.0250x over previous
import functools

import jax
import jax.numpy as jnp
from jax import lax
from jax.experimental import pallas as pl
from jax.experimental.pallas import tpu as pltpu
from jax.experimental.pallas import tpu_sc as plsc

L = 16


def _build_sc_gather(n_b: int, n_l: int, n_rows: int, d: int):
    info = plsc.get_sparse_core_info()
    nc, ns = info.num_cores, info.num_subcores
    nw = nc * ns

    bw = n_b // nw
    assert bw * nw == n_b
    lt_n = n_l // 8
    assert lt_n * 8 == n_l
    bc_w = 256
    bc_n = bw // bc_w
    assert bc_n * bc_w == bw and bc_n % 2 == 0
    pairs = 8 * (bc_w // L)

    mesh = plsc.VectorSubcoreMesh(core_axis_name="c", subcore_axis_name="s")

    @functools.partial(
        pl.kernel,
        mesh=mesh,
        compiler_params=pltpu.CompilerParams(needs_layout_passes=False),
        out_type=jax.ShapeDtypeStruct((d, n_l, n_b), jnp.float32),
        scratch_types=[
            pltpu.VMEM((n_rows * d,), jnp.float32),
            pltpu.VMEM((8, bc_w), jnp.int32),
            pltpu.VMEM((8, bc_w), jnp.int32),
            pltpu.VMEM((d, 8, bc_w), jnp.float32),
            pltpu.VMEM((d, 8, bc_w), jnp.float32),
            pltpu.SemaphoreType.DMA,
            pltpu.SemaphoreType.DMA,
            pltpu.SemaphoreType.DMA,
            pltpu.SemaphoreType.DMA,
        ],
    )
    def sc_gather(idx_hbm, tbl_hbm, out_hbm, tbl_v, idx_v0, idx_v1,
                  out_v0, out_v1, sin0, sin1, sout0, sout1):
        sin = (sin0, sin1)
        sout = (sout0, sout1)
        idx_v = (idx_v0, idx_v1)
        out_v = (out_v0, out_v1)
        wid = lax.axis_index("s") * nc + lax.axis_index("c")
        b0w = wid * bw
        pltpu.sync_copy(tbl_hbm, tbl_v)

        n_chunks = lt_n * bc_n

        def in_slice(ci):
            lt = ci // bc_n
            bc = ci % bc_n
            return idx_hbm.at[pl.ds(lt * 8, 8), pl.ds(b0w + bc * bc_w, bc_w)]

        def out_slice(ci):
            lt = ci // bc_n
            bc = ci % bc_n
            return out_hbm.at[:, pl.ds(lt * 8, 8), pl.ds(b0w + bc * bc_w, bc_w)]

        for b in range(2):
            pltpu.async_copy(in_slice(b), idx_v[b], sin[b])

        def pair_body(cj, _):
            for b in range(2):
                ci = cj * 2 + b
                pltpu.make_async_copy(in_slice(ci), idx_v[b], sin[b]).wait()

                @pl.when(ci >= 2)
                def _wait_out():
                    pltpu.make_async_copy(
                        out_v[b], out_slice(ci - 2), sout[b]
                    ).wait()

                @plsc.parallel_loop(0, pairs, 1, unroll=2)
                def group_body(p):
                    lr = p // (bc_w // L)
                    g = p % (bc_w // L)
                    idx16 = idx_v[b][lr, pl.ds(g * L, L)]
                    addr = idx16 * d
                    for f in range(d):
                        vals = plsc.load_gather(tbl_v, [addr + f])
                        out_v[b][f, lr, pl.ds(g * L, L)] = vals

                pltpu.async_copy(out_v[b], out_slice(ci), sout[b])

                @pl.when(ci + 2 < n_chunks)
                def _next_in():
                    pltpu.async_copy(in_slice(ci + 2), idx_v[b], sin[b])

            return 0

        lax.fori_loop(0, n_chunks // 2, pair_body, 0, unroll=False)

        for b in range(2):
            pltpu.make_async_copy(
                out_v[b], out_slice(n_chunks - 2 + b), sout[b]
            ).wait()

    return sc_gather


def kernel(indices, features):
    b, hl = indices.shape
    v, d = features.shape
    idx_t = jnp.transpose(indices.astype(jnp.int32))
    tbl_flat = features.reshape(-1)
    fn = _build_sc_gather(b, hl, v, d)
    out_t = fn(idx_t, tbl_flat)
    return jnp.transpose(out_t, (2, 1, 0))
